# baseline (speedup 1.0000x reference)
import jax
import jax.numpy as jnp
from jax.experimental import pallas as pl
from jax.experimental.pallas import tpu as pltpu

BT = 512


def _ffn_kernel(x_ref, w1_ref, b1_ref, w2_ref, b2_ref, o_ref):
    h = jnp.dot(x_ref[...], w1_ref[...], preferred_element_type=jnp.float32)
    h = h + b1_ref[...]
    h = h * 0.5 * (1.0 + jax.lax.erf(h * 0.7071067811865476))
    o = jnp.dot(h.astype(jnp.bfloat16), w2_ref[...],
                preferred_element_type=jnp.float32)
    o_ref[...] = o + b2_ref[...]


def kernel(inputs, W1, b1, W2, b2, input_split):
    del input_split
    tokens, d_model = inputs.shape
    d_ff = W1.shape[1]
    x16 = inputs.astype(jnp.bfloat16)
    w1_16 = W1.astype(jnp.bfloat16)
    w2_16 = W2.astype(jnp.bfloat16)
    b1_2d = b1.reshape(1, d_ff)
    b2_2d = b2.reshape(1, d_model)
    return pl.pallas_call(
        _ffn_kernel,
        grid=(tokens // BT,),
        in_specs=[
            pl.BlockSpec((BT, d_model), lambda i: (i, 0)),
            pl.BlockSpec((d_model, d_ff), lambda i: (0, 0)),
            pl.BlockSpec((1, d_ff), lambda i: (0, 0)),
            pl.BlockSpec((d_ff, d_model), lambda i: (0, 0)),
            pl.BlockSpec((1, d_model), lambda i: (0, 0)),
        ],
        out_specs=pl.BlockSpec((BT, d_model), lambda i: (i, 0)),
        out_shape=jax.ShapeDtypeStruct((tokens, d_model), jnp.float32),
        compiler_params=pltpu.CompilerParams(
            dimension_semantics=("arbitrary",),
            vmem_limit_bytes=128 * 1024 * 1024),
    )(x16, w1_16, b1_2d, w2_16, b2_2d)
